# baseline (speedup 1.0000x reference)
import functools

import jax
import jax.numpy as jnp
from jax import lax
from jax.experimental import pallas as pl
from jax.experimental.pallas import tpu as pltpu
from jax.experimental.pallas import tpu_sc as plsc


_TOTAL = 4 * 8192 * 768
_NC, _NS = 2, 16
_NW = _NC * _NS
_PER_W = _TOTAL // _NW
_CHUNK = 49152
_NCH = _PER_W // _CHUNK
_VEC = 16

_mesh = plsc.VectorSubcoreMesh(core_axis_name="c", subcore_axis_name="s")


@functools.partial(
    pl.kernel,
    mesh=_mesh,
    out_type=jax.ShapeDtypeStruct((_TOTAL,), jnp.float32),
    scratch_types=[
        pltpu.VMEM((_CHUNK,), jnp.float32),
        pltpu.SemaphoreType.DMA,
    ],
)
def _sc_double(x_hbm, o_hbm, buf, sem):
    wid = lax.axis_index("s") * _NC + lax.axis_index("c")
    base = wid * _PER_W

    def chunk_body(ci, carry):
        off = base + ci * _CHUNK
        pltpu.async_copy(x_hbm.at[pl.ds(off, _CHUNK)], buf, sem).wait()

        def vec_body(vi, c):
            sl = pl.ds(vi * _VEC, _VEC)
            v = buf[sl]
            buf[sl] = v + v
            return c

        lax.fori_loop(0, _CHUNK // _VEC, vec_body, 0)
        pltpu.async_copy(buf, o_hbm.at[pl.ds(off, _CHUNK)], sem).wait()
        return carry

    lax.fori_loop(0, _NCH, chunk_body, 0)


def kernel(x, Wg, bg, W1, b1, W2, b2):
    out = _sc_double(x.reshape(-1))
    return out.reshape(x.shape)


# speedup vs baseline: 8.7858x; 8.7858x over previous
import jax
import jax.numpy as jnp
from jax.experimental import pallas as pl


_BLOCK_ROWS = 4096


def _double_kernel(x_ref, o_ref):
    o_ref[...] = x_ref[...] + x_ref[...]


def kernel(x, Wg, bg, W1, b1, W2, b2):
    B, T, C = x.shape
    x2 = x.reshape(B * T, C)
    out = pl.pallas_call(
        _double_kernel,
        grid=(B * T // _BLOCK_ROWS,),
        in_specs=[pl.BlockSpec((_BLOCK_ROWS, C), lambda i: (i, 0))],
        out_specs=pl.BlockSpec((_BLOCK_ROWS, C), lambda i: (i, 0)),
        out_shape=jax.ShapeDtypeStruct((B * T, C), x.dtype),
    )(x2)
    return out.reshape(B, T, C)
